# baseline (speedup 1.0000x reference)
import functools

import jax
import jax.numpy as jnp
from jax import lax
from jax.experimental import pallas as pl
from jax.experimental.pallas import tpu as pltpu
from jax.experimental.pallas import tpu_sc as plsc

N_NODES = 10000
HID = 128
HALF = HID // 2
OUT_DIM = 40
NC, NS = 2, 16
NW = NC * NS
K = 40
NBUF = 10
_EPS = 1e-5


def _mesh():
    return plsc.VectorSubcoreMesh(core_axis_name="c", subcore_axis_name="s")


def _deg_call(srcr, zer, one):
    nchunk = srcr.shape[1]

    @functools.partial(
        pl.kernel,
        out_type=(jax.ShapeDtypeStruct((N_NODES,), jnp.float32),
                  jax.ShapeDtypeStruct((N_NODES,), jnp.float32)),
        mesh=_mesh(),
        scratch_types=[
            pltpu.VMEM((nchunk, K), jnp.int32),
            pltpu.VMEM((K,), jnp.float32),
            pltpu.VMEM((1000,), jnp.float32),
            pltpu.VMEM_SHARED((N_NODES,), jnp.float32),
            pltpu.SemaphoreType.DMA,
        ],
    )
    def deg_k(src_hbm, zer_hbm, one_hbm, out0, out1, idx_v, one_v, stage_v,
              acc, ssem):
        c = lax.axis_index("c")
        s = lax.axis_index("s")
        w = c * NS + s
        pltpu.sync_copy(src_hbm.at[w], idx_v)
        pltpu.sync_copy(one_hbm, one_v)

        @pl.when(s < 10)
        def _():
            pltpu.sync_copy(zer_hbm, stage_v)
            pltpu.sync_copy(stage_v, acc.at[pl.ds(s * 1000, 1000)])

        plsc.subcore_barrier()

        def body(j, carry):
            pltpu.async_copy(one_v, acc.at[idx_v.at[j]], ssem, add=True)
            return carry

        lax.fori_loop(0, nchunk, body, 0)

        def drain(j, carry):
            pltpu.make_async_copy(one_v, acc.at[idx_v.at[0]], ssem).wait()
            return carry

        lax.fori_loop(0, nchunk, drain, 0)
        plsc.subcore_barrier()

        @pl.when(s < 10)
        def _():
            pltpu.sync_copy(acc.at[pl.ds(s * 1000, 1000)], stage_v)

            @pl.when(c == 0)
            def _():
                pltpu.sync_copy(stage_v, out0.at[pl.ds(s * 1000, 1000)])

            @pl.when(c == 1)
            def _():
                pltpu.sync_copy(stage_v, out1.at[pl.ds(s * 1000, 1000)])

    return deg_k(srcr, zer, one)


def _mp_call(t0, t1, srcr, dstr, zrows):
    nchunk = srcr.shape[1]

    @functools.partial(
        pl.kernel,
        out_type=(jax.ShapeDtypeStruct((N_NODES, HALF), jnp.float32),
                  jax.ShapeDtypeStruct((N_NODES, HALF), jnp.float32)),
        mesh=_mesh(),
        scratch_types=[
            pltpu.VMEM((nchunk, K), jnp.int32),
            pltpu.VMEM((nchunk, K), jnp.int32),
            pltpu.VMEM((NBUF, K, HALF), jnp.float32),
            pltpu.VMEM((200, HALF), jnp.float32),
            pltpu.VMEM_SHARED((N_NODES, HALF), jnp.float32),
            pltpu.SemaphoreType.DMA((NBUF,)),
            pltpu.SemaphoreType.DMA((NBUF,)),
        ],
        compiler_params=pltpu.CompilerParams(use_tc_tiling_on_sc=False),
    )
    def mp_k(t0_hbm, t1_hbm, src_hbm, dst_hbm, zr_hbm, out0, out1,
             src_v, dst_v, rows_v, stage_v, acc, gsem, ssem):
        c = lax.axis_index("c")
        s = lax.axis_index("s")
        pltpu.sync_copy(src_hbm.at[s], src_v)
        pltpu.sync_copy(dst_hbm.at[s], dst_v)

        @pl.when(s < 10)
        def _():
            for i in range(5):
                pltpu.sync_copy(zr_hbm,
                                acc.at[pl.ds(s * 1000 + i * 200, 200)])

        plsc.subcore_barrier()

        def pipeline(tab_hbm):
            for j in range(NBUF - 1):
                pltpu.async_copy(tab_hbm.at[src_v.at[j]], rows_v.at[j],
                                 gsem.at[j])

            def group(q, carry):
                for b in range(NBUF):
                    jb = q * NBUF + b
                    pltpu.make_async_copy(tab_hbm.at[src_v.at[jb]],
                                          rows_v.at[b], gsem.at[b]).wait()
                    pltpu.async_copy(rows_v.at[b], acc.at[dst_v.at[jb]],
                                     ssem.at[b], add=True)
                    jn = jb + NBUF - 1
                    bn = (b + NBUF - 1) % NBUF

                    @pl.when(jn < nchunk)
                    def _():
                        @pl.when(jn >= NBUF)
                        def _():
                            pltpu.make_async_copy(
                                rows_v.at[bn],
                                acc.at[dst_v.at[jn - NBUF]],
                                ssem.at[bn]).wait()

                        pltpu.async_copy(tab_hbm.at[src_v.at[jn]],
                                         rows_v.at[bn], gsem.at[bn])
                return carry

            lax.fori_loop(0, nchunk // NBUF, group, 0)
            for i in range(NBUF):
                jb = nchunk - NBUF + i
                pltpu.make_async_copy(rows_v.at[jb % NBUF],
                                      acc.at[dst_v.at[jb]],
                                      ssem.at[jb % NBUF]).wait()

        @pl.when(c == 0)
        def _():
            pipeline(t0_hbm)

        @pl.when(c == 1)
        def _():
            pipeline(t1_hbm)

        plsc.subcore_barrier()

        @pl.when(s < 10)
        def _():
            @pl.when(c == 0)
            def _():
                pltpu.sync_copy(acc.at[pl.ds(s * 1000, 1000)],
                                out0.at[pl.ds(s * 1000, 1000)])

            @pl.when(c == 1)
            def _():
                pltpu.sync_copy(acc.at[pl.ds(s * 1000, 1000)],
                                out1.at[pl.ds(s * 1000, 1000)])

    return mp_k(t0, t1, srcr, dstr, zrows)


_R = 1000


def _gelu(v):
    return 0.5 * v * (1.0 + lax.erf(v * 0.7071067811865476))


def _ln(h, g, b):
    mu = jnp.mean(h, axis=-1, keepdims=True)
    var = jnp.mean((h - mu) ** 2, axis=-1, keepdims=True)
    return (h - mu) / jnp.sqrt(var + _EPS) * g[None, :] + b[None, :]


def _dot(a, b):
    return jnp.dot(a, b, preferred_element_type=jnp.float32)


def _row_spec(width):
    return pl.BlockSpec((_R, width), lambda i: (i, 0))


def _deg_spec():
    return pl.BlockSpec((_R, 1), lambda i: (i, 0))


def _full_spec(shape):
    nd = len(shape)
    return pl.BlockSpec(shape, (lambda i: (0,) * nd))


def _split_rp(rp, rp0_ref, rp1_ref):
    rp0_ref[...] = rp[:, :HALF]
    rp1_ref[...] = rp[:, HALF:]


def _tc_in0(x, w_in, b_in):
    def body(x_ref, w_ref, b_ref, h_ref):
        h_ref[...] = _gelu(_dot(x_ref[...], w_ref[...]) + b_ref[...][None, :])

    return pl.pallas_call(
        body,
        grid=(N_NODES // _R,),
        in_specs=[
            _row_spec(x.shape[1]),
            _full_spec(w_in.shape),
            _full_spec(b_in.shape),
        ],
        out_specs=[_row_spec(HID)],
        out_shape=[jax.ShapeDtypeStruct((N_NODES, HID), jnp.float32)],
    )(x, w_in, b_in)[0]


def _tc_in1(h, d0, d1, ln_g, ln_b):
    def body(h_ref, d0_ref, d1_ref, g_ref, bb_ref, rp0_ref, rp1_ref):
        rdeg = lax.rsqrt(d0_ref[...] + d1_ref[...])
        _split_rp(_ln(h_ref[...], g_ref[...], bb_ref[...]) * rdeg,
                  rp0_ref, rp1_ref)

    return pl.pallas_call(
        body,
        grid=(N_NODES // _R,),
        in_specs=[
            _row_spec(HID),
            _deg_spec(),
            _deg_spec(),
            _full_spec(ln_g.shape),
            _full_spec(ln_b.shape),
        ],
        out_specs=[_row_spec(HALF), _row_spec(HALF)],
        out_shape=[jax.ShapeDtypeStruct((N_NODES, HALF), jnp.float32),
                   jax.ShapeDtypeStruct((N_NODES, HALF), jnp.float32)],
    )(h, d0, d1, ln_g, ln_b)


def _tc_layer(h, a0, a1, d0, d1, w1, b1, w2, b2, ln_g, ln_b):
    def body(h_ref, a0_ref, a1_ref, d0_ref, d1_ref, w1_ref, b1_ref,
             w2_ref, b2_ref, g_ref, bb_ref, hout_ref, rp0_ref, rp1_ref):
        rdeg = lax.rsqrt(d0_ref[...] + d1_ref[...])
        agg = jnp.concatenate([a0_ref[...], a1_ref[...]], axis=-1) * rdeg
        f = _gelu(_dot(agg, w1_ref[...]) + b1_ref[...][None, :])
        f = _dot(f, w2_ref[...]) + b2_ref[...][None, :]
        hn = h_ref[...] + f
        hout_ref[...] = hn
        _split_rp(_ln(hn, g_ref[...], bb_ref[...]) * rdeg, rp0_ref, rp1_ref)

    return pl.pallas_call(
        body,
        grid=(N_NODES // _R,),
        in_specs=[
            _row_spec(HID), _row_spec(HALF), _row_spec(HALF),
            _deg_spec(), _deg_spec(),
            _full_spec(w1.shape), _full_spec(b1.shape),
            _full_spec(w2.shape), _full_spec(b2.shape),
            _full_spec(ln_g.shape), _full_spec(ln_b.shape),
        ],
        out_specs=[_row_spec(HID), _row_spec(HALF), _row_spec(HALF)],
        out_shape=[jax.ShapeDtypeStruct((N_NODES, HID), jnp.float32),
                   jax.ShapeDtypeStruct((N_NODES, HALF), jnp.float32),
                   jax.ShapeDtypeStruct((N_NODES, HALF), jnp.float32)],
    )(h, a0, a1, d0, d1, w1, b1, w2, b2, ln_g, ln_b)


def _tc_final(h, a0, a1, d0, d1, w1, b1, w2, b2, og, ob, w_out, b_out):
    def body(h_ref, a0_ref, a1_ref, d0_ref, d1_ref, w1_ref, b1_ref,
             w2_ref, b2_ref, g_ref, bb_ref, wo_ref, bo_ref, out_ref):
        rdeg = lax.rsqrt(d0_ref[...] + d1_ref[...])
        agg = jnp.concatenate([a0_ref[...], a1_ref[...]], axis=-1) * rdeg
        f = _gelu(_dot(agg, w1_ref[...]) + b1_ref[...][None, :])
        f = _dot(f, w2_ref[...]) + b2_ref[...][None, :]
        hn = h_ref[...] + f
        out_ref[...] = (_dot(_ln(hn, g_ref[...], bb_ref[...]), wo_ref[...])
                        + bo_ref[...][None, :])

    return pl.pallas_call(
        body,
        grid=(N_NODES // _R,),
        in_specs=[
            _row_spec(HID), _row_spec(HALF), _row_spec(HALF),
            _deg_spec(), _deg_spec(),
            _full_spec(w1.shape), _full_spec(b1.shape),
            _full_spec(w2.shape), _full_spec(b2.shape),
            _full_spec(og.shape), _full_spec(ob.shape),
            _full_spec(w_out.shape), _full_spec(b_out.shape),
        ],
        out_specs=[_row_spec(OUT_DIM)],
        out_shape=[jax.ShapeDtypeStruct((N_NODES, OUT_DIM), jnp.float32)],
    )(h, a0, a1, d0, d1, w1, b1, w2, b2, og, ob, w_out, b_out)[0]


def kernel(x, edge_index, params):
    src = edge_index[0]
    dst = edge_index[1]
    e = src.shape[0]

    ept_deg = e // NW
    nchunk_deg = ept_deg // K
    srcr_deg = src.reshape(NW, nchunk_deg, K)

    ept = e // NS
    nchunk = ept // K
    srcr = src.reshape(NS, nchunk, K)
    dstr = dst.reshape(NS, nchunk, K)

    zer = jnp.zeros((1000,), jnp.float32)
    one = jnp.ones((K,), jnp.float32)
    zrows = jnp.zeros((200, HALF), jnp.float32)

    d0, d1 = _deg_call(srcr_deg, zer, one)
    d0 = d0.reshape(N_NODES, 1)
    d1 = d1.reshape(N_NODES, 1)

    p = params
    lps = p['layers']
    h = _tc_in0(x, p['w_in'], p['b_in'])
    rp0, rp1 = _tc_in1(h, d0, d1, lps[0]['ln_g'], lps[0]['ln_b'])
    for i in range(len(lps)):
        a0, a1 = _mp_call(rp0, rp1, srcr, dstr, zrows)
        lp = lps[i]
        if i + 1 < len(lps):
            lq = lps[i + 1]
            h, rp0, rp1 = _tc_layer(h, a0, a1, d0, d1, lp['w1'], lp['b1'],
                                    lp['w2'], lp['b2'],
                                    lq['ln_g'], lq['ln_b'])
        else:
            out = _tc_final(h, a0, a1, d0, d1, lp['w1'], lp['b1'],
                            lp['w2'], lp['b2'], p['out_ln_g'], p['out_ln_b'],
                            p['w_out'], p['b_out'])
    return out


# speedup vs baseline: 1.0362x; 1.0362x over previous
import functools

import jax
import jax.numpy as jnp
from jax import lax
from jax.experimental import pallas as pl
from jax.experimental.pallas import tpu as pltpu
from jax.experimental.pallas import tpu_sc as plsc

N_NODES = 10000
HID = 128
HALF = HID // 2
OUT_DIM = 40
NC, NS = 2, 16
NW = NC * NS
K = 40
NBUF = 10
_EPS = 1e-5


def _mesh():
    return plsc.VectorSubcoreMesh(core_axis_name="c", subcore_axis_name="s")


def _deg_call(srcr, zer, one):
    nchunk = srcr.shape[1]

    @functools.partial(
        pl.kernel,
        out_type=(jax.ShapeDtypeStruct((N_NODES,), jnp.float32),
                  jax.ShapeDtypeStruct((N_NODES,), jnp.float32)),
        mesh=_mesh(),
        scratch_types=[
            pltpu.VMEM((nchunk, K), jnp.int32),
            pltpu.VMEM((K,), jnp.float32),
            pltpu.VMEM((1000,), jnp.float32),
            pltpu.VMEM_SHARED((N_NODES,), jnp.float32),
            pltpu.SemaphoreType.DMA,
        ],
    )
    def deg_k(src_hbm, zer_hbm, one_hbm, out0, out1, idx_v, one_v, stage_v,
              acc, ssem):
        c = lax.axis_index("c")
        s = lax.axis_index("s")
        w = c * NS + s
        pltpu.sync_copy(src_hbm.at[w], idx_v)
        pltpu.sync_copy(one_hbm, one_v)

        @pl.when(s < 10)
        def _():
            pltpu.sync_copy(zer_hbm, stage_v)
            pltpu.sync_copy(stage_v, acc.at[pl.ds(s * 1000, 1000)])

        plsc.subcore_barrier()

        def body(j, carry):
            pltpu.async_copy(one_v, acc.at[idx_v.at[j]], ssem, add=True)
            return carry

        lax.fori_loop(0, nchunk, body, 0)

        def drain(j, carry):
            pltpu.make_async_copy(one_v, acc.at[idx_v.at[0]], ssem).wait()
            return carry

        lax.fori_loop(0, nchunk, drain, 0)
        plsc.subcore_barrier()

        @pl.when(s < 10)
        def _():
            pltpu.sync_copy(acc.at[pl.ds(s * 1000, 1000)], stage_v)

            @pl.when(c == 0)
            def _():
                pltpu.sync_copy(stage_v, out0.at[pl.ds(s * 1000, 1000)])

            @pl.when(c == 1)
            def _():
                pltpu.sync_copy(stage_v, out1.at[pl.ds(s * 1000, 1000)])

    return deg_k(srcr, zer, one)


def _mp_call(t0, t1, srcr, dstr, zrows):
    nchunk = srcr.shape[1]

    @functools.partial(
        pl.kernel,
        out_type=(jax.ShapeDtypeStruct((N_NODES, HALF), jnp.float32),
                  jax.ShapeDtypeStruct((N_NODES, HALF), jnp.float32)),
        mesh=_mesh(),
        scratch_types=[
            pltpu.VMEM((nchunk, K), jnp.int32),
            pltpu.VMEM((nchunk, K), jnp.int32),
            pltpu.VMEM((NBUF, K, HALF), jnp.float32),
            pltpu.VMEM_SHARED((N_NODES, HALF), jnp.float32),
            pltpu.SemaphoreType.DMA((NBUF,)),
            pltpu.SemaphoreType.DMA((NBUF,)),
        ],
        compiler_params=pltpu.CompilerParams(use_tc_tiling_on_sc=False),
    )
    def mp_k(t0_hbm, t1_hbm, src_hbm, dst_hbm, zr_hbm, out0, out1,
             src_v, dst_v, rows_v, acc, gsem, ssem):
        c = lax.axis_index("c")
        s = lax.axis_index("s")
        pltpu.sync_copy(src_hbm.at[s], src_v)
        pltpu.sync_copy(dst_hbm.at[s], dst_v)

        @pl.when(s < 10)
        def _():
            for i in range(5):
                pltpu.sync_copy(zr_hbm,
                                acc.at[pl.ds(s * 1000 + i * 200, 200)])

        plsc.subcore_barrier()

        def pipeline(tab_hbm):
            for j in range(NBUF - 1):
                pltpu.async_copy(tab_hbm.at[src_v.at[j]], rows_v.at[j],
                                 gsem.at[j])

            def group(q, carry):
                for b in range(NBUF):
                    jb = q * NBUF + b
                    pltpu.make_async_copy(tab_hbm.at[src_v.at[jb]],
                                          rows_v.at[b], gsem.at[b]).wait()
                    pltpu.async_copy(rows_v.at[b], acc.at[dst_v.at[jb]],
                                     ssem.at[b], add=True)
                    jn = jb + NBUF - 1
                    bn = (b + NBUF - 1) % NBUF

                    @pl.when(jn < nchunk)
                    def _():
                        @pl.when(jn >= NBUF)
                        def _():
                            pltpu.make_async_copy(
                                rows_v.at[bn],
                                acc.at[dst_v.at[jn - NBUF]],
                                ssem.at[bn]).wait()

                        pltpu.async_copy(tab_hbm.at[src_v.at[jn]],
                                         rows_v.at[bn], gsem.at[bn])
                return carry

            lax.fori_loop(0, nchunk // NBUF, group, 0)
            for i in range(NBUF):
                jb = nchunk - NBUF + i
                pltpu.make_async_copy(rows_v.at[jb % NBUF],
                                      acc.at[dst_v.at[jb]],
                                      ssem.at[jb % NBUF]).wait()

        @pl.when(c == 0)
        def _():
            pipeline(t0_hbm)

        @pl.when(c == 1)
        def _():
            pipeline(t1_hbm)

        plsc.subcore_barrier()

        @pl.when(s < 10)
        def _():
            @pl.when(c == 0)
            def _():
                pltpu.sync_copy(acc.at[pl.ds(s * 1000, 1000)],
                                out0.at[pl.ds(s * 1000, 1000)])

            @pl.when(c == 1)
            def _():
                pltpu.sync_copy(acc.at[pl.ds(s * 1000, 1000)],
                                out1.at[pl.ds(s * 1000, 1000)])

    return mp_k(t0, t1, srcr, dstr, zrows)


_R = 2000


def _gelu(v):
    return 0.5 * v * (1.0 + lax.erf(v * 0.7071067811865476))


def _ln(h, g, b):
    mu = jnp.mean(h, axis=-1, keepdims=True)
    var = jnp.mean((h - mu) ** 2, axis=-1, keepdims=True)
    return (h - mu) / jnp.sqrt(var + _EPS) * g[None, :] + b[None, :]


def _dot(a, b):
    return jnp.dot(a, b, preferred_element_type=jnp.float32)


def _row_spec(width):
    return pl.BlockSpec((_R, width), lambda i: (i, 0))


def _deg_spec():
    return pl.BlockSpec((_R, 1), lambda i: (i, 0))


def _full_spec(shape):
    nd = len(shape)
    return pl.BlockSpec(shape, (lambda i: (0,) * nd))


def _split_rp(rp, rp0_ref, rp1_ref):
    rp0_ref[...] = rp[:, :HALF]
    rp1_ref[...] = rp[:, HALF:]


def _tc_in0(x, w_in, b_in):
    def body(x_ref, w_ref, b_ref, h_ref):
        h_ref[...] = _gelu(_dot(x_ref[...], w_ref[...]) + b_ref[...][None, :])

    return pl.pallas_call(
        body,
        grid=(N_NODES // _R,),
        in_specs=[
            _row_spec(x.shape[1]),
            _full_spec(w_in.shape),
            _full_spec(b_in.shape),
        ],
        out_specs=[_row_spec(HID)],
        out_shape=[jax.ShapeDtypeStruct((N_NODES, HID), jnp.float32)],
    )(x, w_in, b_in)[0]


def _tc_in1(h, d0, d1, ln_g, ln_b):
    def body(h_ref, d0_ref, d1_ref, g_ref, bb_ref, rp0_ref, rp1_ref):
        rdeg = lax.rsqrt(d0_ref[...] + d1_ref[...])
        _split_rp(_ln(h_ref[...], g_ref[...], bb_ref[...]) * rdeg,
                  rp0_ref, rp1_ref)

    return pl.pallas_call(
        body,
        grid=(N_NODES // _R,),
        in_specs=[
            _row_spec(HID),
            _deg_spec(),
            _deg_spec(),
            _full_spec(ln_g.shape),
            _full_spec(ln_b.shape),
        ],
        out_specs=[_row_spec(HALF), _row_spec(HALF)],
        out_shape=[jax.ShapeDtypeStruct((N_NODES, HALF), jnp.float32),
                   jax.ShapeDtypeStruct((N_NODES, HALF), jnp.float32)],
    )(h, d0, d1, ln_g, ln_b)


def _tc_layer(h, a0, a1, d0, d1, w1, b1, w2, b2, ln_g, ln_b):
    def body(h_ref, a0_ref, a1_ref, d0_ref, d1_ref, w1_ref, b1_ref,
             w2_ref, b2_ref, g_ref, bb_ref, hout_ref, rp0_ref, rp1_ref):
        rdeg = lax.rsqrt(d0_ref[...] + d1_ref[...])
        agg = jnp.concatenate([a0_ref[...], a1_ref[...]], axis=-1) * rdeg
        f = _gelu(_dot(agg, w1_ref[...]) + b1_ref[...][None, :])
        f = _dot(f, w2_ref[...]) + b2_ref[...][None, :]
        hn = h_ref[...] + f
        hout_ref[...] = hn
        _split_rp(_ln(hn, g_ref[...], bb_ref[...]) * rdeg, rp0_ref, rp1_ref)

    return pl.pallas_call(
        body,
        grid=(N_NODES // _R,),
        in_specs=[
            _row_spec(HID), _row_spec(HALF), _row_spec(HALF),
            _deg_spec(), _deg_spec(),
            _full_spec(w1.shape), _full_spec(b1.shape),
            _full_spec(w2.shape), _full_spec(b2.shape),
            _full_spec(ln_g.shape), _full_spec(ln_b.shape),
        ],
        out_specs=[_row_spec(HID), _row_spec(HALF), _row_spec(HALF)],
        out_shape=[jax.ShapeDtypeStruct((N_NODES, HID), jnp.float32),
                   jax.ShapeDtypeStruct((N_NODES, HALF), jnp.float32),
                   jax.ShapeDtypeStruct((N_NODES, HALF), jnp.float32)],
    )(h, a0, a1, d0, d1, w1, b1, w2, b2, ln_g, ln_b)


def _tc_final(h, a0, a1, d0, d1, w1, b1, w2, b2, og, ob, w_out, b_out):
    def body(h_ref, a0_ref, a1_ref, d0_ref, d1_ref, w1_ref, b1_ref,
             w2_ref, b2_ref, g_ref, bb_ref, wo_ref, bo_ref, out_ref):
        rdeg = lax.rsqrt(d0_ref[...] + d1_ref[...])
        agg = jnp.concatenate([a0_ref[...], a1_ref[...]], axis=-1) * rdeg
        f = _gelu(_dot(agg, w1_ref[...]) + b1_ref[...][None, :])
        f = _dot(f, w2_ref[...]) + b2_ref[...][None, :]
        hn = h_ref[...] + f
        out_ref[...] = (_dot(_ln(hn, g_ref[...], bb_ref[...]), wo_ref[...])
                        + bo_ref[...][None, :])

    return pl.pallas_call(
        body,
        grid=(N_NODES // _R,),
        in_specs=[
            _row_spec(HID), _row_spec(HALF), _row_spec(HALF),
            _deg_spec(), _deg_spec(),
            _full_spec(w1.shape), _full_spec(b1.shape),
            _full_spec(w2.shape), _full_spec(b2.shape),
            _full_spec(og.shape), _full_spec(ob.shape),
            _full_spec(w_out.shape), _full_spec(b_out.shape),
        ],
        out_specs=[_row_spec(OUT_DIM)],
        out_shape=[jax.ShapeDtypeStruct((N_NODES, OUT_DIM), jnp.float32)],
    )(h, a0, a1, d0, d1, w1, b1, w2, b2, og, ob, w_out, b_out)[0]


def kernel(x, edge_index, params):
    src = edge_index[0]
    dst = edge_index[1]
    e = src.shape[0]

    ept_deg = e // NW
    nchunk_deg = ept_deg // K
    srcr_deg = src.reshape(NW, nchunk_deg, K)

    ept = e // NS
    nchunk = ept // K
    srcr = src.reshape(NS, nchunk, K)
    dstr = dst.reshape(NS, nchunk, K)

    zer = jnp.zeros((1000,), jnp.float32)
    one = jnp.ones((K,), jnp.float32)
    zrows = jnp.zeros((200, HALF), jnp.float32)

    d0, d1 = _deg_call(srcr_deg, zer, one)
    d0 = d0.reshape(N_NODES, 1)
    d1 = d1.reshape(N_NODES, 1)

    p = params
    lps = p['layers']
    h = _tc_in0(x, p['w_in'], p['b_in'])
    rp0, rp1 = _tc_in1(h, d0, d1, lps[0]['ln_g'], lps[0]['ln_b'])
    for i in range(len(lps)):
        a0, a1 = _mp_call(rp0, rp1, srcr, dstr, zrows)
        lp = lps[i]
        if i + 1 < len(lps):
            lq = lps[i + 1]
            h, rp0, rp1 = _tc_layer(h, a0, a1, d0, d1, lp['w1'], lp['b1'],
                                    lp['w2'], lp['b2'],
                                    lq['ln_g'], lq['ln_b'])
        else:
            out = _tc_final(h, a0, a1, d0, d1, lp['w1'], lp['b1'],
                            lp['w2'], lp['b2'], p['out_ln_g'], p['out_ln_b'],
                            p['w_out'], p['b_out'])
    return out
